# baseline (speedup 1.0000x reference)
import functools

import jax
import jax.numpy as jnp
from jax import lax
from jax.experimental import pallas as pl
from jax.experimental.pallas import tpu as pltpu
from jax.experimental.pallas import tpu_sc as plsc

N = 10000
E = 160000
W = 32
WP = 128
KW = 256
KIN = 4
DEPTH = 6

NC, NS = 2, 16
NW = NC * NS
CHUNK = 128
RPW = 40
ROWS_P = NW * RPW
E_PAD = ROWS_P * CHUNK
GG = 4
GS = 2
NP_ = 10240
NPW = NP_ // NS

_sc_mesh = plsc.VectorSubcoreMesh(
    core_axis_name="c", subcore_axis_name="s", num_cores=NC, num_subcores=NS)


@functools.partial(
    pl.kernel,
    out_type=jax.ShapeDtypeStruct((E_PAD, WP), jnp.float32),
    mesh=_sc_mesh,
    scratch_types=[pltpu.VMEM((GG, CHUNK), jnp.int32),
                   pltpu.VMEM((GG * CHUNK, WP), jnp.float32),
                   pltpu.SemaphoreType.DMA],
)
def _sc_gather(h_hbm, src_hbm, out_hbm, idx_v, rows_v, sem):
    wid = lax.axis_index("s") * NC + lax.axis_index("c")
    base = wid * RPW

    def body(g, carry):
        r0 = base + g * GG
        pltpu.sync_copy(src_hbm.at[pl.ds(r0, GG)], idx_v)
        descs = [
            pltpu.async_copy(h_hbm.at[idx_v.at[t]],
                             rows_v.at[pl.ds(t * CHUNK, CHUNK)], sem)
            for t in range(GG)
        ]
        for d in descs:
            d.wait()
        pltpu.sync_copy(rows_v, out_hbm.at[pl.ds(r0 * CHUNK, GG * CHUNK)])
        return carry

    lax.fori_loop(0, RPW // GG, body, 0)


@functools.partial(
    pl.kernel,
    out_type=jax.ShapeDtypeStruct((NC * NP_, WP), jnp.float32),
    mesh=_sc_mesh,
    scratch_types=[pltpu.VMEM((CHUNK,), jnp.int32),
                   pltpu.VMEM((CHUNK, WP), jnp.float32),
                   pltpu.SemaphoreType.DMA,
                   pltpu.VMEM_SHARED((NP_, WP), jnp.float32)],
)
def _sc_scatter(msg_hbm, dst_hbm, zeros_hbm, out_hbm, idx_v, rows_v, sem, acc_s):
    c = lax.axis_index("c")
    s = lax.axis_index("s")
    wid = s * NC + c
    base = wid * RPW
    pltpu.sync_copy(zeros_hbm, acc_s.at[pl.ds(s * NPW, NPW)])
    plsc.subcore_barrier()

    def body(g, carry):
        r = base + g
        pltpu.sync_copy(dst_hbm.at[r], idx_v)
        pltpu.sync_copy(msg_hbm.at[pl.ds(r * CHUNK, CHUNK)], rows_v)
        pltpu.sync_copy(rows_v, acc_s.at[idx_v], add=True)
        return carry

    lax.fori_loop(0, RPW, body, 0)
    plsc.subcore_barrier()
    pltpu.sync_copy(acc_s.at[pl.ds(s * NPW, NPW)],
                    out_hbm.at[pl.ds(c * NP_ + s * NPW, NPW)])


@functools.partial(
    pl.kernel,
    out_type=jax.ShapeDtypeStruct((NC * NP_, WP), jnp.float32),
    mesh=_sc_mesh,
    scratch_types=[pltpu.VMEM((CHUNK,), jnp.int32),
                   pltpu.VMEM((CHUNK, WP), jnp.float32),
                   pltpu.SemaphoreType.DMA,
                   pltpu.VMEM_SHARED((NP_, WP), jnp.float32)],
)
def _sc_degree(dst_hbm, ones_hbm, zeros_hbm, out_hbm, idx_v, ones_v, sem, acc_s):
    c = lax.axis_index("c")
    s = lax.axis_index("s")
    wid = s * NC + c
    base = wid * RPW
    pltpu.sync_copy(zeros_hbm, acc_s.at[pl.ds(s * NPW, NPW)])
    pltpu.sync_copy(ones_hbm, ones_v)
    plsc.subcore_barrier()

    def body(g, carry):
        r = base + g
        pltpu.sync_copy(dst_hbm.at[r], idx_v)
        pltpu.sync_copy(ones_v, acc_s.at[idx_v], add=True)
        return carry

    lax.fori_loop(0, RPW, body, 0)
    plsc.subcore_barrier()
    pltpu.sync_copy(acc_s.at[pl.ds(s * NPW, NPW)],
                    out_hbm.at[pl.ds(c * NP_ + s * NPW, NPW)])


EB = 2048


def _mlp_body(ea, k1t, k1b, k2t, k2b, k3t, k3b, out):
    a = jnp.maximum(jnp.dot(ea[...], k1t[...],
                            preferred_element_type=jnp.float32) + k1b[...], 0.0)
    a = jnp.maximum(jnp.dot(a, k2t[...],
                            preferred_element_type=jnp.float32) + k2b[...], 0.0)
    out[...] = jnp.dot(a, k3t[...],
                       preferred_element_type=jnp.float32) + k3b[...]


def _edge_mlp(edge_attr, k1t, k1b, k2t, k2b, k3t, k3b):
    grid = (E_PAD // EB,)
    return pl.pallas_call(
        _mlp_body,
        grid=grid,
        in_specs=[
            pl.BlockSpec((EB, KIN), lambda i: (i, 0)),
            pl.BlockSpec((KIN, KW), lambda i: (0, 0)),
            pl.BlockSpec((1, KW), lambda i: (0, 0)),
            pl.BlockSpec((KW, KW), lambda i: (0, 0)),
            pl.BlockSpec((1, KW), lambda i: (0, 0)),
            pl.BlockSpec((KW, W * W), lambda i: (0, 0)),
            pl.BlockSpec((1, W * W), lambda i: (0, 0)),
        ],
        out_specs=pl.BlockSpec((EB, W * W), lambda i: (i, 0)),
        out_shape=jax.ShapeDtypeStruct((E_PAD, W * W), jnp.float32),
    )(edge_attr, k1t, k1b, k2t, k2b, k3t, k3b)


def _matvec_body(xj, we, tile, out):
    x = xj[...][:, :W]
    t = tile[...]
    x1 = x.astype(jnp.bfloat16).astype(jnp.float32)
    r1 = x - x1
    x2 = r1.astype(jnp.bfloat16).astype(jnp.float32)
    x3 = r1 - x2
    xrep = (jnp.dot(x1, t, preferred_element_type=jnp.float32)
            + jnp.dot(x2, t, preferred_element_type=jnp.float32)
            + jnp.dot(x3, t, preferred_element_type=jnp.float32))
    w = we[...]
    acc = w[:, 0:128] * xrep[:, 0:128]
    for k in range(1, 8):
        acc = acc + w[:, 128 * k:128 * (k + 1)] * xrep[:, 128 * k:128 * (k + 1)]
    msg = acc[:, 0:32] + acc[:, 32:64] + acc[:, 64:96] + acc[:, 96:128]
    out[...] = jnp.concatenate(
        [msg, jnp.zeros((EB, WP - W), jnp.float32)], axis=1)


def _matvec(xj, we, tile):
    grid = (E_PAD // EB,)
    return pl.pallas_call(
        _matvec_body,
        grid=grid,
        in_specs=[
            pl.BlockSpec((EB, WP), lambda i: (i, 0)),
            pl.BlockSpec((EB, W * W), lambda i: (i, 0)),
            pl.BlockSpec((W, W * W), lambda i: (0, 0)),
        ],
        out_specs=pl.BlockSpec((EB, WP), lambda i: (i, 0)),
        out_shape=jax.ShapeDtypeStruct((E_PAD, WP), jnp.float32),
    )(xj, we, tile)


def _prologue_body(x, fw, fb, degp, h0, invd):
    h = x[...] * fw[...] + fb[...]
    h0[...] = jnp.concatenate(
        [h, jnp.zeros((N, WP - W), jnp.float32)], axis=1)
    d = degp[...]
    deg = d[:N, :1] + d[NP_:NP_ + N, :1]
    invd[...] = jnp.broadcast_to(1.0 / jnp.maximum(deg, 1.0), (N, W))


def _prologue(x, fw, fb, degp):
    return pl.pallas_call(
        _prologue_body,
        out_shape=[jax.ShapeDtypeStruct((N, WP), jnp.float32),
                   jax.ShapeDtypeStruct((N, W), jnp.float32)],
    )(x, fw, fb, degp)


def _update_body(p, invd, h, root, b, out):
    pp = p[...]
    agg = (pp[:N, :W] + pp[NP_:NP_ + N, :W]) * invd[...]
    hn = jnp.maximum(
        agg + jnp.dot(h[...][:, :W], root[...],
                      preferred_element_type=jnp.float32) + b[...], 0.0)
    out[...] = jnp.concatenate(
        [hn, jnp.zeros((N, WP - W), jnp.float32)], axis=1)


def _update(aggp, invd, h, root, b):
    return pl.pallas_call(
        _update_body,
        out_shape=jax.ShapeDtypeStruct((N, WP), jnp.float32),
    )(aggp, invd, h, root, b)


def _epilogue_body(h, fw, fb, out):
    out[...] = jnp.dot(h[...][:, :W], fw[...],
                       preferred_element_type=jnp.float32) + fb[...]


def _epilogue(h, fw, fb):
    return pl.pallas_call(
        _epilogue_body,
        out_shape=jax.ShapeDtypeStruct((N, 1), jnp.float32),
    )(h, fw, fb)


def kernel(x, edge_attr, fc1_w, fc1_b, k1_w, k1_b, k2_w, k2_b, k3_w, k3_b,
           root, conv_bias, fc2_w, fc2_b, edge_index):
    ei = edge_index.astype(jnp.int32)
    src2d = jnp.pad(ei[0], (0, E_PAD - E)).reshape(ROWS_P, CHUNK)
    dst2d = jnp.pad(ei[1], (0, E_PAD - E),
                    constant_values=N).reshape(ROWS_P, CHUNK)
    ea_p = jnp.pad(edge_attr, ((0, E_PAD - E), (0, 0)))
    zeros = jnp.zeros((NPW, WP), jnp.float32)
    ones = jnp.ones((CHUNK, WP), jnp.float32)
    tile_i = jnp.repeat(jnp.eye(W, dtype=jnp.float32), W, axis=1)

    we = _edge_mlp(ea_p, k1_w.T, k1_b.reshape(1, KW),
                   k2_w.T, k2_b.reshape(1, KW),
                   k3_w.T, k3_b.reshape(1, W * W))
    degp = _sc_degree(dst2d, ones, zeros)
    h, invd = _prologue(x, fc1_w.T, fc1_b.reshape(1, W), degp)
    cb = conv_bias.reshape(1, W)
    for _ in range(DEPTH):
        xj = _sc_gather(h, src2d)
        msg = _matvec(xj, we, tile_i)
        aggp = _sc_scatter(msg, dst2d, zeros)
        h = _update(aggp, invd, h, root, cb)
    return _epilogue(h, fc2_w.T, fc2_b.reshape(1, 1))


# speedup vs baseline: 1.3335x; 1.3335x over previous
import functools

import jax
import jax.numpy as jnp
from jax import lax
from jax.experimental import pallas as pl
from jax.experimental.pallas import tpu as pltpu
from jax.experimental.pallas import tpu_sc as plsc

N = 10000
E = 160000
W = 32
WP = 128
KW = 256
KIN = 4
DEPTH = 6

NC, NS = 2, 16
NW = NC * NS
CHUNK = 128
RPW = 40
ROWS_P = NW * RPW
E_PAD = ROWS_P * CHUNK
GG = 4
GS = 2
NP_ = 10240
NPW = NP_ // NS

_sc_mesh = plsc.VectorSubcoreMesh(
    core_axis_name="c", subcore_axis_name="s", num_cores=NC, num_subcores=NS)


@functools.partial(
    pl.kernel,
    out_type=jax.ShapeDtypeStruct((E_PAD, WP), jnp.float32),
    mesh=_sc_mesh,
    scratch_types=[pltpu.VMEM((GG, CHUNK), jnp.int32),
                   pltpu.VMEM((GG * CHUNK, WP), jnp.float32),
                   pltpu.SemaphoreType.DMA],
)
def _sc_gather(h_hbm, src_hbm, out_hbm, idx_v, rows_v, sem):
    wid = lax.axis_index("s") * NC + lax.axis_index("c")
    base = wid * RPW

    def body(g, carry):
        r0 = base + g * GG
        pltpu.sync_copy(src_hbm.at[pl.ds(r0, GG)], idx_v)
        descs = [
            pltpu.async_copy(h_hbm.at[idx_v.at[t]],
                             rows_v.at[pl.ds(t * CHUNK, CHUNK)], sem)
            for t in range(GG)
        ]
        for d in descs:
            d.wait()
        pltpu.sync_copy(rows_v, out_hbm.at[pl.ds(r0 * CHUNK, GG * CHUNK)])
        return carry

    lax.fori_loop(0, RPW // GG, body, 0)


@functools.partial(
    pl.kernel,
    out_type=jax.ShapeDtypeStruct((NC * NP_, WP), jnp.float32),
    mesh=_sc_mesh,
    scratch_types=[pltpu.VMEM((CHUNK,), jnp.int32),
                   pltpu.VMEM((CHUNK, WP), jnp.float32),
                   pltpu.SemaphoreType.DMA,
                   pltpu.VMEM_SHARED((NP_, WP), jnp.float32)],
)
def _sc_scatter(msg_hbm, dst_hbm, zeros_hbm, out_hbm, idx_v, rows_v, sem, acc_s):
    c = lax.axis_index("c")
    s = lax.axis_index("s")
    wid = s * NC + c
    base = wid * RPW
    pltpu.sync_copy(zeros_hbm, acc_s.at[pl.ds(s * NPW, NPW)])
    plsc.subcore_barrier()

    def body(g, carry):
        r = base + g
        pltpu.sync_copy(dst_hbm.at[r], idx_v)
        pltpu.sync_copy(msg_hbm.at[pl.ds(r * CHUNK, CHUNK)], rows_v)
        pltpu.sync_copy(rows_v, acc_s.at[idx_v], add=True)
        return carry

    lax.fori_loop(0, RPW, body, 0)
    plsc.subcore_barrier()
    pltpu.sync_copy(acc_s.at[pl.ds(s * NPW, NPW)],
                    out_hbm.at[pl.ds(c * NP_ + s * NPW, NPW)])


@functools.partial(
    pl.kernel,
    out_type=jax.ShapeDtypeStruct((NC * NP_, WP), jnp.float32),
    mesh=_sc_mesh,
    scratch_types=[pltpu.VMEM((CHUNK,), jnp.int32),
                   pltpu.VMEM((CHUNK, WP), jnp.float32),
                   pltpu.SemaphoreType.DMA,
                   pltpu.VMEM_SHARED((NP_, WP), jnp.float32)],
)
def _sc_degree(dst_hbm, ones_hbm, zeros_hbm, out_hbm, idx_v, ones_v, sem, acc_s):
    c = lax.axis_index("c")
    s = lax.axis_index("s")
    wid = s * NC + c
    base = wid * RPW
    pltpu.sync_copy(zeros_hbm, acc_s.at[pl.ds(s * NPW, NPW)])
    pltpu.sync_copy(ones_hbm, ones_v)
    plsc.subcore_barrier()

    def body(g, carry):
        r = base + g
        pltpu.sync_copy(dst_hbm.at[r], idx_v)
        pltpu.sync_copy(ones_v, acc_s.at[idx_v], add=True)
        return carry

    lax.fori_loop(0, RPW, body, 0)
    plsc.subcore_barrier()
    pltpu.sync_copy(acc_s.at[pl.ds(s * NPW, NPW)],
                    out_hbm.at[pl.ds(c * NP_ + s * NPW, NPW)])


EB = 2048


def _mlp_body(ea, k1t, k1b, k2t, k2b, k3t, k3b, out):
    a = jnp.maximum(jnp.dot(ea[...], k1t[...],
                            preferred_element_type=jnp.float32) + k1b[...], 0.0)
    a = jnp.maximum(jnp.dot(a, k2t[...],
                            preferred_element_type=jnp.float32) + k2b[...], 0.0)
    out[...] = jnp.dot(a, k3t[...],
                       preferred_element_type=jnp.float32) + k3b[...]


def _edge_mlp(edge_attr, k1t, k1b, k2t, k2b, k3t, k3b):
    grid = (E_PAD // EB,)
    return pl.pallas_call(
        _mlp_body,
        grid=grid,
        in_specs=[
            pl.BlockSpec((EB, KIN), lambda i: (i, 0)),
            pl.BlockSpec((KIN, KW), lambda i: (0, 0)),
            pl.BlockSpec((1, KW), lambda i: (0, 0)),
            pl.BlockSpec((KW, KW), lambda i: (0, 0)),
            pl.BlockSpec((1, KW), lambda i: (0, 0)),
            pl.BlockSpec((KW, W * W), lambda i: (0, 0)),
            pl.BlockSpec((1, W * W), lambda i: (0, 0)),
        ],
        out_specs=pl.BlockSpec((EB, W * W), lambda i: (i, 0)),
        out_shape=jax.ShapeDtypeStruct((E_PAD, W * W), jnp.float32),
    )(edge_attr, k1t, k1b, k2t, k2b, k3t, k3b)


def _matvec_body(xj, we, tile, out):
    x = xj[...][:, :W]
    t = tile[...]
    x1 = x.astype(jnp.bfloat16).astype(jnp.float32)
    r1 = x - x1
    x2 = r1.astype(jnp.bfloat16).astype(jnp.float32)
    x3 = r1 - x2
    xrep = (jnp.dot(x1, t, preferred_element_type=jnp.float32)
            + jnp.dot(x2, t, preferred_element_type=jnp.float32)
            + jnp.dot(x3, t, preferred_element_type=jnp.float32))
    w = we[...]
    acc = w[:, 0:128] * xrep[:, 0:128]
    for k in range(1, 8):
        acc = acc + w[:, 128 * k:128 * (k + 1)] * xrep[:, 128 * k:128 * (k + 1)]
    msg = acc[:, 0:32] + acc[:, 32:64] + acc[:, 64:96] + acc[:, 96:128]
    out[...] = jnp.concatenate(
        [msg, jnp.zeros((EB, WP - W), jnp.float32)], axis=1)


def _matvec(xj, we, tile):
    grid = (E_PAD // EB,)
    return pl.pallas_call(
        _matvec_body,
        grid=grid,
        in_specs=[
            pl.BlockSpec((EB, WP), lambda i: (i, 0)),
            pl.BlockSpec((EB, W * W), lambda i: (i, 0)),
            pl.BlockSpec((W, W * W), lambda i: (0, 0)),
        ],
        out_specs=pl.BlockSpec((EB, WP), lambda i: (i, 0)),
        out_shape=jax.ShapeDtypeStruct((E_PAD, WP), jnp.float32),
    )(xj, we, tile)


def _prologue_body(x, fw, fb, degp, h0, invd):
    h = x[...] * fw[...] + fb[...]
    h0[...] = jnp.concatenate(
        [h, jnp.zeros((N, WP - W), jnp.float32)], axis=1)
    d = degp[...]
    deg = d[:N, :1] + d[NP_:NP_ + N, :1]
    invd[...] = jnp.broadcast_to(1.0 / jnp.maximum(deg, 1.0), (N, W))


def _prologue(x, fw, fb, degp):
    return pl.pallas_call(
        _prologue_body,
        out_shape=[jax.ShapeDtypeStruct((N, WP), jnp.float32),
                   jax.ShapeDtypeStruct((N, W), jnp.float32)],
    )(x, fw, fb, degp)


def _update_body(p, invd, h, root, b, out):
    pp = p[...]
    agg = (pp[:N, :W] + pp[NP_:NP_ + N, :W]) * invd[...]
    hn = jnp.maximum(
        agg + jnp.dot(h[...][:, :W], root[...],
                      preferred_element_type=jnp.float32) + b[...], 0.0)
    out[...] = jnp.concatenate(
        [hn, jnp.zeros((N, WP - W), jnp.float32)], axis=1)


def _update(aggp, invd, h, root, b):
    return pl.pallas_call(
        _update_body,
        out_shape=jax.ShapeDtypeStruct((N, WP), jnp.float32),
    )(aggp, invd, h, root, b)


def _epilogue_body(h, fw, fb, out):
    out[...] = jnp.dot(h[...][:, :W], fw[...],
                       preferred_element_type=jnp.float32) + fb[...]


def _epilogue(h, fw, fb):
    return pl.pallas_call(
        _epilogue_body,
        out_shape=jax.ShapeDtypeStruct((N, 1), jnp.float32),
    )(h, fw, fb)


def kernel(x, edge_attr, fc1_w, fc1_b, k1_w, k1_b, k2_w, k2_b, k3_w, k3_b,
           root, conv_bias, fc2_w, fc2_b, edge_index):
    ei = edge_index.astype(jnp.int32)
    pad_src = jnp.arange(E_PAD - E, dtype=jnp.int32) % N
    pad_dst = N + jnp.arange(E_PAD - E, dtype=jnp.int32) % (NP_ - N)
    src2d = jnp.concatenate([ei[0], pad_src]).reshape(ROWS_P, CHUNK)
    dst2d = jnp.concatenate([ei[1], pad_dst]).reshape(ROWS_P, CHUNK)
    ea_p = jnp.pad(edge_attr, ((0, E_PAD - E), (0, 0)))
    zeros = jnp.zeros((NPW, WP), jnp.float32)
    ones = jnp.ones((CHUNK, WP), jnp.float32)
    tile_i = jnp.repeat(jnp.eye(W, dtype=jnp.float32), W, axis=1)

    we = _edge_mlp(ea_p, k1_w.T, k1_b.reshape(1, KW),
                   k2_w.T, k2_b.reshape(1, KW),
                   k3_w.T, k3_b.reshape(1, W * W))
    degp = _sc_degree(dst2d, ones, zeros)
    h, invd = _prologue(x, fc1_w.T, fc1_b.reshape(1, W), degp)
    cb = conv_bias.reshape(1, W)
    for _ in range(DEPTH):
        xj = _sc_gather(h, src2d)
        msg = _matvec(xj, we, tile_i)
        aggp = _sc_scatter(msg, dst2d, zeros)
        h = _update(aggp, invd, h, root, cb)
    return _epilogue(h, fc2_w.T, fc2_b.reshape(1, 1))


# speedup vs baseline: 1.3908x; 1.0429x over previous
import functools

import jax
import jax.numpy as jnp
from jax import lax
from jax.experimental import pallas as pl
from jax.experimental.pallas import tpu as pltpu
from jax.experimental.pallas import tpu_sc as plsc

N = 10000
E = 160000
W = 32
WP = 128
KW = 256
KIN = 4
DEPTH = 6

NC, NS = 2, 16
NW = NC * NS
CHUNK = 128
RPW = 40
ROWS_P = NW * RPW
E_PAD = ROWS_P * CHUNK
GG = 4
GS = 2
NP_ = 10240
NPW = NP_ // NS

_sc_mesh = plsc.VectorSubcoreMesh(
    core_axis_name="c", subcore_axis_name="s", num_cores=NC, num_subcores=NS)


@functools.partial(
    pl.kernel,
    out_type=jax.ShapeDtypeStruct((E_PAD, WP), jnp.float32),
    mesh=_sc_mesh,
    scratch_types=[pltpu.VMEM((GG, CHUNK), jnp.int32),
                   pltpu.VMEM((GG * CHUNK, WP), jnp.float32),
                   pltpu.SemaphoreType.DMA],
)
def _sc_gather(h_hbm, src_hbm, out_hbm, idx_v, rows_v, sem):
    wid = lax.axis_index("s") * NC + lax.axis_index("c")
    base = wid * RPW

    def body(g, carry):
        r0 = base + g * GG
        pltpu.sync_copy(src_hbm.at[pl.ds(r0, GG)], idx_v)
        descs = [
            pltpu.async_copy(h_hbm.at[idx_v.at[t]],
                             rows_v.at[pl.ds(t * CHUNK, CHUNK)], sem)
            for t in range(GG)
        ]
        for d in descs:
            d.wait()
        pltpu.sync_copy(rows_v, out_hbm.at[pl.ds(r0 * CHUNK, GG * CHUNK)])
        return carry

    lax.fori_loop(0, RPW // GG, body, 0)


@functools.partial(
    pl.kernel,
    out_type=jax.ShapeDtypeStruct((NC * NP_, WP), jnp.float32),
    mesh=_sc_mesh,
    scratch_types=[pltpu.VMEM((CHUNK,), jnp.int32),
                   pltpu.VMEM((CHUNK,), jnp.int32),
                   pltpu.VMEM((CHUNK, WP), jnp.float32),
                   pltpu.VMEM((CHUNK, WP), jnp.float32),
                   pltpu.SemaphoreType.DMA,
                   pltpu.SemaphoreType.DMA,
                   pltpu.VMEM_SHARED((NP_, WP), jnp.float32)],
)
def _sc_scatter(msg_hbm, dst_hbm, zeros_hbm, out_hbm, idx0, idx1, rows0,
                rows1, sem0, sem1, acc_s):
    c = lax.axis_index("c")
    s = lax.axis_index("s")
    wid = s * NC + c
    base = wid * RPW
    idx = (idx0, idx1)
    rows = (rows0, rows1)
    sems = (sem0, sem1)
    pltpu.sync_copy(zeros_hbm, acc_s.at[pl.ds(s * NPW, NPW)])
    plsc.subcore_barrier()

    pltpu.sync_copy(dst_hbm.at[base], idx0)
    pltpu.async_copy(msg_hbm.at[pl.ds(base * CHUNK, CHUNK)], rows0, sem0)

    def body(gg, carry):
        for b in range(2):
            g = gg * 2 + b
            r = base + g
            pltpu.make_async_copy(
                msg_hbm.at[pl.ds(r * CHUNK, CHUNK)], rows[b], sems[b]).wait()

            @pl.when(g + 1 < RPW)
            def _():
                pltpu.sync_copy(dst_hbm.at[r + 1], idx[1 - b])
                pltpu.async_copy(msg_hbm.at[pl.ds((r + 1) * CHUNK, CHUNK)],
                                 rows[1 - b], sems[1 - b])
            pltpu.sync_copy(rows[b], acc_s.at[idx[b]], add=True)
        return carry

    lax.fori_loop(0, RPW // 2, body, 0)
    plsc.subcore_barrier()
    pltpu.sync_copy(acc_s.at[pl.ds(s * NPW, NPW)],
                    out_hbm.at[pl.ds(c * NP_ + s * NPW, NPW)])


@functools.partial(
    pl.kernel,
    out_type=jax.ShapeDtypeStruct((NC * NP_, WP), jnp.float32),
    mesh=_sc_mesh,
    scratch_types=[pltpu.VMEM((CHUNK,), jnp.int32),
                   pltpu.VMEM((CHUNK, WP), jnp.float32),
                   pltpu.SemaphoreType.DMA,
                   pltpu.VMEM_SHARED((NP_, WP), jnp.float32)],
)
def _sc_degree(dst_hbm, ones_hbm, zeros_hbm, out_hbm, idx_v, ones_v, sem, acc_s):
    c = lax.axis_index("c")
    s = lax.axis_index("s")
    wid = s * NC + c
    base = wid * RPW
    pltpu.sync_copy(zeros_hbm, acc_s.at[pl.ds(s * NPW, NPW)])
    pltpu.sync_copy(ones_hbm, ones_v)
    plsc.subcore_barrier()

    def body(g, carry):
        r = base + g
        pltpu.sync_copy(dst_hbm.at[r], idx_v)
        pltpu.sync_copy(ones_v, acc_s.at[idx_v], add=True)
        return carry

    lax.fori_loop(0, RPW, body, 0)
    plsc.subcore_barrier()
    pltpu.sync_copy(acc_s.at[pl.ds(s * NPW, NPW)],
                    out_hbm.at[pl.ds(c * NP_ + s * NPW, NPW)])


EB = 2048


def _mlp_body(ea, k1t, k1b, k2t, k2b, k3t, k3b, out):
    a = jnp.maximum(jnp.dot(ea[...], k1t[...],
                            preferred_element_type=jnp.float32) + k1b[...], 0.0)
    a = jnp.maximum(jnp.dot(a, k2t[...],
                            preferred_element_type=jnp.float32) + k2b[...], 0.0)
    out[...] = jnp.dot(a, k3t[...],
                       preferred_element_type=jnp.float32) + k3b[...]


def _edge_mlp(edge_attr, k1t, k1b, k2t, k2b, k3t, k3b):
    grid = (E_PAD // EB,)
    return pl.pallas_call(
        _mlp_body,
        grid=grid,
        in_specs=[
            pl.BlockSpec((EB, KIN), lambda i: (i, 0)),
            pl.BlockSpec((KIN, KW), lambda i: (0, 0)),
            pl.BlockSpec((1, KW), lambda i: (0, 0)),
            pl.BlockSpec((KW, KW), lambda i: (0, 0)),
            pl.BlockSpec((1, KW), lambda i: (0, 0)),
            pl.BlockSpec((KW, W * W), lambda i: (0, 0)),
            pl.BlockSpec((1, W * W), lambda i: (0, 0)),
        ],
        out_specs=pl.BlockSpec((EB, W * W), lambda i: (i, 0)),
        out_shape=jax.ShapeDtypeStruct((E_PAD, W * W), jnp.float32),
    )(edge_attr, k1t, k1b, k2t, k2b, k3t, k3b)


def _matvec_body(xj, we, tile, out):
    x = xj[...][:, :W]
    t = tile[...]
    x1 = x.astype(jnp.bfloat16).astype(jnp.float32)
    r1 = x - x1
    x2 = r1.astype(jnp.bfloat16).astype(jnp.float32)
    x3 = r1 - x2
    xrep = (jnp.dot(x1, t, preferred_element_type=jnp.float32)
            + jnp.dot(x2, t, preferred_element_type=jnp.float32)
            + jnp.dot(x3, t, preferred_element_type=jnp.float32))
    w = we[...]
    acc = w[:, 0:128] * xrep[:, 0:128]
    for k in range(1, 8):
        acc = acc + w[:, 128 * k:128 * (k + 1)] * xrep[:, 128 * k:128 * (k + 1)]
    msg = acc[:, 0:32] + acc[:, 32:64] + acc[:, 64:96] + acc[:, 96:128]
    out[...] = jnp.concatenate(
        [msg, jnp.zeros((EB, WP - W), jnp.float32)], axis=1)


def _matvec(xj, we, tile):
    grid = (E_PAD // EB,)
    return pl.pallas_call(
        _matvec_body,
        grid=grid,
        in_specs=[
            pl.BlockSpec((EB, WP), lambda i: (i, 0)),
            pl.BlockSpec((EB, W * W), lambda i: (i, 0)),
            pl.BlockSpec((W, W * W), lambda i: (0, 0)),
        ],
        out_specs=pl.BlockSpec((EB, WP), lambda i: (i, 0)),
        out_shape=jax.ShapeDtypeStruct((E_PAD, WP), jnp.float32),
    )(xj, we, tile)


def _prologue_body(x, fw, fb, degp, h0, invd):
    h = x[...] * fw[...] + fb[...]
    h0[...] = jnp.concatenate(
        [h, jnp.zeros((N, WP - W), jnp.float32)], axis=1)
    d = degp[...]
    deg = d[:N, :1] + d[NP_:NP_ + N, :1]
    invd[...] = jnp.broadcast_to(1.0 / jnp.maximum(deg, 1.0), (N, W))


def _prologue(x, fw, fb, degp):
    return pl.pallas_call(
        _prologue_body,
        out_shape=[jax.ShapeDtypeStruct((N, WP), jnp.float32),
                   jax.ShapeDtypeStruct((N, W), jnp.float32)],
    )(x, fw, fb, degp)


def _update_body(p, invd, h, root, b, out):
    pp = p[...]
    agg = (pp[:N, :W] + pp[NP_:NP_ + N, :W]) * invd[...]
    hn = jnp.maximum(
        agg + jnp.dot(h[...][:, :W], root[...],
                      preferred_element_type=jnp.float32) + b[...], 0.0)
    out[...] = jnp.concatenate(
        [hn, jnp.zeros((N, WP - W), jnp.float32)], axis=1)


def _update(aggp, invd, h, root, b):
    return pl.pallas_call(
        _update_body,
        out_shape=jax.ShapeDtypeStruct((N, WP), jnp.float32),
    )(aggp, invd, h, root, b)


def _epilogue_body(h, fw, fb, out):
    out[...] = jnp.dot(h[...][:, :W], fw[...],
                       preferred_element_type=jnp.float32) + fb[...]


def _epilogue(h, fw, fb):
    return pl.pallas_call(
        _epilogue_body,
        out_shape=jax.ShapeDtypeStruct((N, 1), jnp.float32),
    )(h, fw, fb)


def kernel(x, edge_attr, fc1_w, fc1_b, k1_w, k1_b, k2_w, k2_b, k3_w, k3_b,
           root, conv_bias, fc2_w, fc2_b, edge_index):
    ei = edge_index.astype(jnp.int32)
    pad_src = jnp.arange(E_PAD - E, dtype=jnp.int32) % N
    pad_dst = N + jnp.arange(E_PAD - E, dtype=jnp.int32) % (NP_ - N)
    src2d = jnp.concatenate([ei[0], pad_src]).reshape(ROWS_P, CHUNK)
    dst2d = jnp.concatenate([ei[1], pad_dst]).reshape(ROWS_P, CHUNK)
    ea_p = jnp.pad(edge_attr, ((0, E_PAD - E), (0, 0)))
    zeros = jnp.zeros((NPW, WP), jnp.float32)
    ones = jnp.ones((CHUNK, WP), jnp.float32)
    tile_i = jnp.repeat(jnp.eye(W, dtype=jnp.float32), W, axis=1)

    we = _edge_mlp(ea_p, k1_w.T, k1_b.reshape(1, KW),
                   k2_w.T, k2_b.reshape(1, KW),
                   k3_w.T, k3_b.reshape(1, W * W))
    degp = _sc_degree(dst2d, ones, zeros)
    h, invd = _prologue(x, fc1_w.T, fc1_b.reshape(1, W), degp)
    cb = conv_bias.reshape(1, W)
    for _ in range(DEPTH):
        xj = _sc_gather(h, src2d)
        msg = _matvec(xj, we, tile_i)
        aggp = _sc_scatter(msg, dst2d, zeros)
        h = _update(aggp, invd, h, root, cb)
    return _epilogue(h, fc2_w.T, fc2_b.reshape(1, 1))
